# Initial kernel scaffold; baseline (speedup 1.0000x reference)
import jax
import jax.numpy as jnp
from jax.experimental import pallas as pl


def kernel(x, v_idx, e_idx, restriction_maps, step_size, bias):
    raise NotImplementedError("write your pallas kernel here")


# speedup vs baseline: 2.2901x; 2.2901x over previous
import dataclasses

import jax
import jax.numpy as jnp
from jax import lax
from jax.experimental import pallas as pl
from jax.experimental.pallas import tpu as pltpu
from jax.experimental.pallas import tpu_sc as plsc

N = 10000
M = 10000
NNZ = 160000
D = 128
DE = 4
NUM_STEPS = 4

W = 128
KT = 640


def _vmesh():
    return plsc.VectorSubcoreMesh(core_axis_name="core", subcore_axis_name="subcore")


def _sc_params():
    cp = pltpu.CompilerParams()
    if "needs_layout_passes" in pltpu.CompilerParams.__dataclass_fields__:
        cp = dataclasses.replace(cp, needs_layout_passes=False)
    return cp


def _sc_gather_rows(table, idx3):
    dv = table.shape[1]
    nw = idx3.shape[0]

    @pl.kernel(
        out_type=jax.ShapeDtypeStruct((nw * W, dv), table.dtype),
        mesh=_vmesh(),
    )
    def k(tab_hbm, i_hbm, o_hbm):
        def body(i_vmem, o_vmem):
            pltpu.sync_copy(tab_hbm.at[i_vmem.at[0, 0]], o_vmem)

        pltpu.emit_pipeline(
            body,
            grid=(nw,),
            in_specs=[pl.BlockSpec((1, 1, W), index_map=lambda i: (i, 0, 0))],
            out_specs=[pl.BlockSpec((W, dv), index_map=lambda i: (i, 0))],
            core_axis_name=("core", "subcore"),
            dimension_semantics=(pltpu.PARALLEL,),
        )(i_hbm, o_hbm)

    return k(table, idx3)


def _iota16():
    return lax.iota(jnp.int32, 16)


def _sc_gather_small(tab_flat, idx3):
    t4 = tab_flat.shape[0]
    nw = idx3.shape[0]

    @pl.kernel(
        out_type=jax.ShapeDtypeStruct((nw * W * DE,), jnp.float32),
        mesh=_vmesh(),
        scratch_types=[pltpu.VMEM((t4,), jnp.float32)],
        compiler_params=_sc_params(),
    )
    def k(tab_hbm, i_hbm, o_hbm, tabv):
        pltpu.sync_copy(tab_hbm, tabv)

        def body(i_vmem, o_vmem):
            for g in range(W // 16):
                e4 = i_vmem[0, 0, pl.ds(16 * g, 16)] * 4
                pos = _iota16() * 4 + (64 * g)
                for j in range(DE):
                    vals = plsc.load_gather(tabv, [e4 + j])
                    plsc.store_scatter(o_vmem, [pos + j], vals)

        pltpu.emit_pipeline(
            body,
            grid=(nw,),
            in_specs=[pl.BlockSpec((1, 1, W), index_map=lambda i: (i, 0, 0))],
            out_specs=[pl.BlockSpec((W * DE,), index_map=lambda i: (i,))],
            core_axis_name=("core", "subcore"),
            dimension_semantics=(pltpu.PARALLEL,),
        )(i_hbm, o_hbm)

    return k(tab_flat, idx3)


NW_SC = 32


def _sc_scatter_add_small(upd_flat, idx3, t4):
    nw = idx3.shape[0]

    @pl.kernel(
        out_type=jax.ShapeDtypeStruct((NW_SC, t4), jnp.float32),
        mesh=_vmesh(),
        scratch_types=[pltpu.VMEM((t4,), jnp.float32)],
        compiler_params=_sc_params(),
    )
    def k(u_hbm, i_hbm, z_hbm, o_hbm, tabv):
        cid = lax.axis_index("core")
        sid = lax.axis_index("subcore")
        wid = sid * 2 + cid
        pltpu.sync_copy(z_hbm, tabv)

        def body(u_vmem, i_vmem):
            for g in range(W // 16):
                e4 = i_vmem[0, 0, pl.ds(16 * g, 16)] * 4
                pos = _iota16() * 4 + (64 * g)
                for j in range(DE):
                    vals = plsc.load_gather(u_vmem, [pos + j])
                    plsc.addupdate_scatter(tabv, [e4 + j], vals)

        pltpu.emit_pipeline(
            body,
            grid=(nw,),
            in_specs=[
                pl.BlockSpec((W * DE,), index_map=lambda i: (i,)),
                pl.BlockSpec((1, 1, W), index_map=lambda i: (i, 0, 0)),
            ],
            out_specs=[],
            core_axis_name=("core", "subcore"),
            dimension_semantics=(pltpu.PARALLEL,),
        )(u_hbm, i_hbm)

        pltpu.sync_copy(tabv, o_hbm.at[wid])

    return k(upd_flat, idx3, jnp.zeros((t4,), jnp.float32))


def _sc_scatter_add_rows(upd, idx3, zeros_tab):
    t, dv = zeros_tab.shape
    nw = idx3.shape[0]

    @pl.kernel(
        out_type=jax.ShapeDtypeStruct((2, t, dv), jnp.float32),
        mesh=_vmesh(),
        scratch_types=[pltpu.VMEM_SHARED((t, dv), jnp.float32)],
    )
    def k(u_hbm, i_hbm, z_hbm, o_hbm, acc):
        cid = lax.axis_index("core")
        sid = lax.axis_index("subcore")

        @pl.when(sid == 0)
        def _():
            pltpu.sync_copy(z_hbm, acc)

        plsc.subcore_barrier()

        def body(u_vmem, i_vmem):
            pltpu.sync_copy(u_vmem, acc.at[i_vmem.at[0, 0]], add=True)

        pltpu.emit_pipeline(
            body,
            grid=(nw,),
            in_specs=[
                pl.BlockSpec((W, dv), index_map=lambda i: (i, 0)),
                pl.BlockSpec((1, 1, W), index_map=lambda i: (i, 0, 0)),
            ],
            out_specs=[],
            core_axis_name=("core", "subcore"),
            dimension_semantics=(pltpu.PARALLEL,),
        )(u_hbm, i_hbm)

        plsc.subcore_barrier()

        @pl.when(sid == 0)
        def _():
            pltpu.sync_copy(acc, o_hbm.at[cid])

    return k(upd, idx3, zeros_tab)


def _tc_z_pass(rt2, x_inc):
    g = NNZ // KT
    h = KT // 2

    def body(rt_ref, x_ref, o_ref):
        xe = jnp.repeat(x_ref[...].reshape(h, 2, D), DE, axis=1)
        o_ref[...] = jnp.sum(rt_ref[...] * xe, axis=-1)

    return pl.pallas_call(
        body,
        grid=(g,),
        in_specs=[
            pl.BlockSpec((h, 2 * DE, D), lambda i: (i, 0, 0)),
            pl.BlockSpec((KT, D), lambda i: (i, 0)),
        ],
        out_specs=pl.BlockSpec((h, 2 * DE), lambda i: (i, 0)),
        out_shape=jax.ShapeDtypeStruct((NNZ // 2, 2 * DE), jnp.float32),
    )(rt2, x_inc)


def _tc_delta_pass(rt2, z, zeg):
    g = NNZ // KT
    h = KT // 2

    def body(rt_ref, z_ref, ze_ref, o_ref):
        dif = z_ref[...] - ze_ref[...]
        dif3 = jnp.broadcast_to(dif[:, :, None], (h, 2 * DE, D))
        prod = rt_ref[...] * dif3
        s = jnp.sum(prod.reshape(h, 2, DE, D), axis=2)
        o_ref[...] = s.reshape(KT, D)

    return pl.pallas_call(
        body,
        grid=(g,),
        in_specs=[
            pl.BlockSpec((h, 2 * DE, D), lambda i: (i, 0, 0)),
            pl.BlockSpec((h, 2 * DE), lambda i: (i, 0)),
            pl.BlockSpec((h, 2 * DE), lambda i: (i, 0)),
        ],
        out_specs=pl.BlockSpec((KT, D), lambda i: (i, 0)),
        out_shape=jax.ShapeDtypeStruct((NNZ, D), jnp.float32),
    )(rt2, z, zeg)


_MR = M // 8
_MC = 32


def _tc_inv_deg(degp):
    def body(d_ref, o_ref):
        d = jnp.sum(d_ref[...], axis=0)
        o_ref[...] = jnp.where(d > 0.0, 1.0 / jnp.where(d > 0.0, d, 1.0), 1.0)

    return pl.pallas_call(
        body,
        out_shape=jax.ShapeDtypeStruct((_MR, _MC), jnp.float32),
    )(degp)


def _tc_edge_mean(zsp, inv4):
    def body(a_ref, b_ref, o_ref):
        o_ref[...] = jnp.sum(a_ref[...], axis=0) * b_ref[...]

    return pl.pallas_call(
        body,
        out_shape=jax.ShapeDtypeStruct((_MR, _MC), jnp.float32),
    )(zsp, inv4)


def _tc_node_update(x, dp, stepb, biasrow):
    bn = 400
    g = N // bn

    def body(x_ref, d_ref, s_ref, b_ref, o_ref):
        o_ref[...] = (
            x_ref[...]
            - s_ref[...] * (d_ref[0] + d_ref[1])
            + b_ref[...]
        )

    return pl.pallas_call(
        body,
        grid=(g,),
        in_specs=[
            pl.BlockSpec((bn, D), lambda i: (i, 0)),
            pl.BlockSpec((2, bn, D), lambda i: (0, i, 0)),
            pl.BlockSpec((1, D), lambda i: (0, 0)),
            pl.BlockSpec((1, D), lambda i: (0, 0)),
        ],
        out_specs=pl.BlockSpec((bn, D), lambda i: (i, 0)),
        out_shape=jax.ShapeDtypeStruct((N, D), jnp.float32),
    )(x, dp, stepb, biasrow)


_DBG_XLA_GATHER_X = False
_DBG_XLA_GATHER_E = False
_DBG_XLA_SCAT_DEG = False
_DBG_XLA_SCAT_ZS = False
_DBG_XLA_SCAT_DELTA = False


def _xla_scatter(upd, idx, zeros_tab):
    p = zeros_tab.at[idx].add(upd)
    return jnp.stack([p, jnp.zeros_like(p)])


def _xla_scatter_small(upd, idx, t):
    p = jnp.zeros((t, DE), jnp.float32).at[idx].add(upd.reshape(-1, DE))
    return jnp.concatenate(
        [p.reshape(1, t * DE), jnp.zeros((NW_SC - 1, t * DE), jnp.float32)], axis=0
    )


def kernel(x, v_idx, e_idx, restriction_maps, step_size, bias):
    v_idx = v_idx.astype(jnp.int32)
    e_idx = e_idx.astype(jnp.int32)
    x = x.astype(jnp.float32)

    rt2 = jnp.swapaxes(restriction_maps, 1, 2).reshape(NNZ // 2, 2 * DE, D)

    vi2 = v_idx.reshape(NNZ // W, 1, W)
    ei2 = e_idx.reshape(NNZ // W, 1, W)
    zeros_m = jnp.zeros((M, DE), jnp.float32)
    zeros_n = jnp.zeros((N, D), jnp.float32)
    ones_upd = jnp.ones((NNZ, DE), jnp.float32)
    stepb = jnp.full((1, D), step_size, jnp.float32)
    biasrow = bias.reshape(1, D).astype(jnp.float32)

    if _DBG_XLA_SCAT_DEG:
        degp = _xla_scatter_small(ones_upd, e_idx, M)
    else:
        degp = _sc_scatter_add_small(ones_upd.reshape(NNZ * DE), ei2, M * DE)
    inv4 = _tc_inv_deg(degp.reshape(NW_SC, _MR, _MC))

    for _ in range(NUM_STEPS):
        if _DBG_XLA_GATHER_X:
            x_inc = x[v_idx]
        else:
            x_inc = _sc_gather_rows(x, vi2)
        z = _tc_z_pass(rt2, x_inc)
        if _DBG_XLA_SCAT_ZS:
            zsp = _xla_scatter_small(z.reshape(NNZ, DE), e_idx, M)
        else:
            zsp = _sc_scatter_add_small(z.reshape(NNZ * DE), ei2, M * DE)
        z_e = _tc_edge_mean(zsp.reshape(NW_SC, _MR, _MC), inv4)
        if _DBG_XLA_GATHER_E:
            zeg = z_e.reshape(M, DE)[e_idx]
        else:
            zeg = _sc_gather_small(z_e.reshape(M * DE), ei2).reshape(NNZ, DE)
        dinc = _tc_delta_pass(rt2, z, zeg.reshape(NNZ // 2, 2 * DE))
        if _DBG_XLA_SCAT_DELTA:
            dp = _xla_scatter(dinc, v_idx, zeros_n)
        else:
            dp = _sc_scatter_add_rows(dinc, vi2, zeros_n)
        x = _tc_node_update(x, dp, stepb, biasrow)

    return x
